# baseline (speedup 1.0000x reference)
import jax
import jax.numpy as jnp
from jax import lax
from jax.experimental import pallas as pl
from jax.experimental.pallas import tpu as pltpu
from jax.experimental.pallas import tpu_sc as plsc

_NUM_ROWS = 10
_EMB_DIM = 5
_L = 16
_NC, _NS = 2, 16
_NW = _NC * _NS
_ROWS, _COLS = 16384, 200
_ROWS_PER_W = _ROWS // _NW
_VPR = _COLS // _L
_TAIL_OFF = _COLS - _L
_TAIL_DUP = _VPR * _L - _TAIL_OFF
_NACC = 4
_CR = 64
_RUNROLL = 2
_NCHUNK = _ROWS_PER_W // _CR


def _sc_body(idx_hbm, w_hbm, out_hbm, idx_v0, idx_v1, w_v, acc_v,
             sem0, sem1):
    wid = lax.axis_index("s") * _NC + lax.axis_index("c")
    r0 = wid * _ROWS_PER_W
    bufs, sems = (idx_v0, idx_v1), (sem0, sem1)

    pltpu.sync_copy(w_hbm, w_v)
    rs = w_v[pl.ds(0, _L)]
    for dcol in range(1, _EMB_DIM):
        rs = rs + w_v[pl.ds(dcol * _L, _L)]

    tail_keep = lax.iota(jnp.int32, _L) >= _TAIL_DUP

    def make_body(buf):
        def body(i, accs):
            out = list(accs)
            n = 0
            for rr in range(_RUNROLL):
                r = i * _RUNROLL + rr
                for u in range(_VPR):
                    v = buf[r, pl.ds(u * _L, _L)]
                    out[n % _NACC] = out[n % _NACC] + rs.at[v].get(
                        mode="promise_in_bounds")
                    n += 1
                vt = buf[r, pl.ds(_TAIL_OFF, _L)]
                g = rs.at[vt].get(mode="promise_in_bounds")
                out[n % _NACC] = out[n % _NACC] + jnp.where(
                    tail_keep, g, 0.0)
                n += 1
            return tuple(out)
        return body

    def start(c):
        return pltpu.async_copy(
            idx_hbm.at[pl.ds(r0 + c * _CR, _CR)], bufs[c % 2], sems[c % 2])

    cps = [None] * _NCHUNK
    cps[0] = start(0)
    cps[1] = start(1)
    zero = jnp.zeros((_L,), jnp.float32)
    accs = (zero,) * _NACC
    for c in range(_NCHUNK):
        cps[c].wait()
        accs = lax.fori_loop(0, _CR // _RUNROLL, make_body(bufs[c % 2]), accs)
        if c + 2 < _NCHUNK:
            cps[c + 2] = start(c + 2)
    acc = accs[0]
    for u in range(1, _NACC):
        acc = acc + accs[u]
    acc_v[...] = acc
    pltpu.sync_copy(acc_v, out_hbm.at[wid])


_sc_call = pl.kernel(
    _sc_body,
    out_type=jax.ShapeDtypeStruct((_NW, _L), jnp.float32),
    mesh=plsc.VectorSubcoreMesh(core_axis_name="c", subcore_axis_name="s"),
    scratch_types=[
        pltpu.VMEM((_CR, _COLS), jnp.int32),
        pltpu.VMEM((_CR, _COLS), jnp.int32),
        pltpu.VMEM((_L * _L,), jnp.float32),
        pltpu.VMEM((_L,), jnp.float32),
        pltpu.SemaphoreType.DMA,
        pltpu.SemaphoreType.DMA,
    ],
)


def _sum_body(x_ref, o_ref):
    o_ref[...] = jnp.sum(x_ref[...], axis=(0, 1), keepdims=True)


def _final_sum(x):
    return pl.pallas_call(
        _sum_body,
        out_shape=jax.ShapeDtypeStruct((1, 1), jnp.float32),
    )(x)[0, 0]


def kernel(indices, weight):
    w_pad = (jnp.zeros((_L, _L), jnp.float32)
             .at[:_EMB_DIM, :_NUM_ROWS].set(weight.T).reshape(-1))
    partials = _sc_call(indices, w_pad)
    return _final_sum(partials)


# speedup vs baseline: 1.0014x; 1.0014x over previous
import jax
import jax.numpy as jnp
from jax import lax
from jax.experimental import pallas as pl
from jax.experimental.pallas import tpu as pltpu
from jax.experimental.pallas import tpu_sc as plsc

_NUM_ROWS = 10
_EMB_DIM = 5
_L = 16
_NC, _NS = 2, 16
_NW = _NC * _NS
_ROWS, _COLS = 16384, 200
_ROWS_PER_W = _ROWS // _NW
_VPR = _COLS // _L
_TAIL_OFF = _COLS - _L
_TAIL_DUP = _VPR * _L - _TAIL_OFF
_NACC = 4
_CR = 64
_RUNROLL = 4
_NCHUNK = _ROWS_PER_W // _CR


def _sc_body(idx_hbm, w_hbm, out_hbm, idx_v0, idx_v1, w_v, acc_v,
             sem0, sem1):
    wid = lax.axis_index("s") * _NC + lax.axis_index("c")
    r0 = wid * _ROWS_PER_W
    bufs, sems = (idx_v0, idx_v1), (sem0, sem1)

    def start(c):
        return pltpu.async_copy(
            idx_hbm.at[pl.ds(r0 + c * _CR, _CR)], bufs[c % 2], sems[c % 2])

    cps = [None] * _NCHUNK
    cps[0] = start(0)
    cps[1] = start(1)

    pltpu.sync_copy(w_hbm, w_v)
    rs = w_v[pl.ds(0, _L)]
    for dcol in range(1, _EMB_DIM):
        rs = rs + w_v[pl.ds(dcol * _L, _L)]

    tail_keep = lax.iota(jnp.int32, _L) >= _TAIL_DUP

    def make_body(buf):
        def body(i, accs):
            out = list(accs)
            n = 0
            for rr in range(_RUNROLL):
                r = i * _RUNROLL + rr
                for u in range(_VPR):
                    v = buf[r, pl.ds(u * _L, _L)]
                    out[n % _NACC] = out[n % _NACC] + rs.at[v].get(
                        mode="promise_in_bounds")
                    n += 1
                vt = buf[r, pl.ds(_TAIL_OFF, _L)]
                g = rs.at[vt].get(mode="promise_in_bounds")
                out[n % _NACC] = out[n % _NACC] + jnp.where(
                    tail_keep, g, 0.0)
                n += 1
            return tuple(out)
        return body

    zero = jnp.zeros((_L,), jnp.float32)
    accs = (zero,) * _NACC
    for c in range(_NCHUNK):
        cps[c].wait()
        accs = lax.fori_loop(
            0, _CR // _RUNROLL, make_body(bufs[c % 2]), accs)
        if c + 2 < _NCHUNK:
            cps[c + 2] = start(c + 2)
    acc = accs[0]
    for u in range(1, _NACC):
        acc = acc + accs[u]
    acc_v[...] = acc
    pltpu.sync_copy(acc_v, out_hbm.at[wid])


_sc_call = pl.kernel(
    _sc_body,
    out_type=jax.ShapeDtypeStruct((_NW, _L), jnp.float32),
    mesh=plsc.VectorSubcoreMesh(core_axis_name="c", subcore_axis_name="s"),
    scratch_types=[
        pltpu.VMEM((_CR, _COLS), jnp.int32),
        pltpu.VMEM((_CR, _COLS), jnp.int32),
        pltpu.VMEM((_L * _L,), jnp.float32),
        pltpu.VMEM((_L,), jnp.float32),
        pltpu.SemaphoreType.DMA,
        pltpu.SemaphoreType.DMA,
    ],
)


def _sum_body(x_ref, o_ref):
    o_ref[...] = jnp.sum(x_ref[...], axis=(0, 1), keepdims=True)


def _final_sum(x):
    return pl.pallas_call(
        _sum_body,
        out_shape=jax.ShapeDtypeStruct((1, 1), jnp.float32),
    )(x)[0, 0]


def kernel(indices, weight):
    w_pad = (jnp.zeros((_L, _L), jnp.float32)
             .at[:_EMB_DIM, :_NUM_ROWS].set(weight.T).reshape(-1))
    partials = _sc_call(indices, w_pad)
    return _final_sum(partials)


# speedup vs baseline: 1.5505x; 1.5484x over previous
import jax
import jax.numpy as jnp
from jax import lax
from jax.experimental import pallas as pl
from jax.experimental.pallas import tpu as pltpu
from jax.experimental.pallas import tpu_sc as plsc

_NUM_ROWS = 10
_EMB_DIM = 5
_L = 16
_NC, _NS = 2, 16
_NW = _NC * _NS
_ROWS, _COLS = 16384, 200
_CPW = _ROWS // _NW
_CC = 128
_NCHUNK = _CPW // _CC
_VPR = _CC // _L
_NACC = 4
_RUNROLL = 2


def _sc_body(idx_hbm, w_hbm, out_hbm, idx_v0, idx_v1, w_v, acc_v,
             sem0, sem1):
    wid = lax.axis_index("s") * _NC + lax.axis_index("c")
    c0 = wid * _CPW
    bufs, sems = (idx_v0, idx_v1), (sem0, sem1)

    def start(c):
        return pltpu.async_copy(
            idx_hbm.at[:, pl.ds(c0 + c * _CC, _CC)], bufs[c % 2],
            sems[c % 2])

    cps = [None] * _NCHUNK
    cps[0] = start(0)
    cps[1] = start(1)

    pltpu.sync_copy(w_hbm, w_v)
    rs = w_v[pl.ds(0, _L)]
    for dcol in range(1, _EMB_DIM):
        rs = rs + w_v[pl.ds(dcol * _L, _L)]

    def make_body(buf):
        def body(i, accs):
            out = list(accs)
            n = 0
            for rr in range(_RUNROLL):
                r = i * _RUNROLL + rr
                for u in range(_VPR):
                    v = buf[r, pl.ds(u * _L, _L)]
                    out[n % _NACC] = out[n % _NACC] + rs.at[v].get(
                        mode="promise_in_bounds")
                    n += 1
            return tuple(out)
        return body

    zero = jnp.zeros((_L,), jnp.float32)
    accs = (zero,) * _NACC
    for c in range(_NCHUNK):
        cps[c].wait()
        accs = lax.fori_loop(
            0, _COLS // _RUNROLL, make_body(bufs[c % 2]), accs)
        if c + 2 < _NCHUNK:
            cps[c + 2] = start(c + 2)
    acc = accs[0]
    for u in range(1, _NACC):
        acc = acc + accs[u]
    acc_v[...] = acc
    pltpu.sync_copy(acc_v, out_hbm.at[wid])


_sc_call = pl.kernel(
    _sc_body,
    out_type=jax.ShapeDtypeStruct((_NW, _L), jnp.float32),
    mesh=plsc.VectorSubcoreMesh(core_axis_name="c", subcore_axis_name="s"),
    scratch_types=[
        pltpu.VMEM((_COLS, _CC), jnp.int32),
        pltpu.VMEM((_COLS, _CC), jnp.int32),
        pltpu.VMEM((_L * _L,), jnp.float32),
        pltpu.VMEM((_L,), jnp.float32),
        pltpu.SemaphoreType.DMA,
        pltpu.SemaphoreType.DMA,
    ],
)


def _sum_body(x_ref, o_ref):
    o_ref[...] = jnp.sum(x_ref[...], axis=(0, 1), keepdims=True)


def _final_sum(x):
    return jnp.reshape(pl.pallas_call(
        _sum_body,
        out_shape=jax.ShapeDtypeStruct((1, 1), jnp.float32),
    )(x), ())


def kernel(indices, weight):
    w_pad = (jnp.zeros((_L, _L), jnp.float32)
             .at[:_EMB_DIM, :_NUM_ROWS].set(weight.T).reshape(-1))
    partials = _sc_call(indices.T, w_pad)
    return _final_sum(partials)
